# Initial kernel scaffold; baseline (speedup 1.0000x reference)
import jax
import jax.numpy as jnp
from jax.experimental import pallas as pl


def kernel(x, action, Wm, bm, W1, b1, W2, b2):
    raise NotImplementedError("write your pallas kernel here")


# speedup vs baseline: 12.6001x; 12.6001x over previous
import jax
import jax.numpy as jnp
from jax.experimental import pallas as pl
from jax.experimental.pallas import tpu as pltpu


def _ct_body(ids_ref, x_ref, act_ref, wm_ref, bm_ref, gmdb_ref, gd_ref,
             w1c0_ref, b10_ref, w20_ref, w1ce_ref, b1e_ref, w2e_ref,
             out_ref, *, lp):
    xb = x_ref[0]

    act = act_ref[0]
    im = jax.nn.sigmoid(
        jnp.dot(act, wm_ref[...], preferred_element_type=jnp.float32)
        + bm_ref[...])
    v = jnp.sum(xb * im, axis=1, keepdims=True)
    ldm = (jnp.log(jnp.maximum(v, 1e-4))
           - jnp.log(jnp.maximum(1.0 - v, 1e-4)))
    mask2d = ldm > gmdb_ref[0]

    def expert(w1c, b1r, w2r):
        r = jnp.dot(xb, w1c, preferred_element_type=jnp.float32)
        a = r[:, :lp] + b1r
        b = r[:, lp:]
        h = a[:, None, :] + b[None, :, :]
        h = jnp.where(h >= 0, h, 0.01 * h)
        s = jnp.sum(h * w2r[None, :, :], axis=-1)
        return jax.nn.sigmoid(s)

    adj0 = expert(w1c0_ref[...], b10_ref[...], w20_ref[...])
    adje = expert(w1ce_ref[0], b1e_ref[0], w2e_ref[0])
    adj = jnp.where(mask2d, adje, adj0)

    ld = (jnp.log(jnp.maximum(adj, 1e-4))
          - jnp.log(jnp.maximum(1.0 - adj, 1e-4)))
    out_ref[0] = (ld > gd_ref[0]).astype(jnp.float32)


def kernel(x, action, Wm, bm, W1, b1, W2, b2):
    Bsz, N, D = x.shape
    A = action.shape[1]
    L = W1.shape[2]
    LP = ((L + 1 + 127) // 128) * 128

    f32 = jnp.float32

    mkey = jax.random.key(42)
    u0 = jax.random.uniform(jax.random.fold_in(mkey, 0), (Bsz, N, 2),
                            minval=1e-10, maxval=1.0)
    g0 = -jnp.log(-jnp.log(u0) + 1e-10)
    gmd = g0[..., 0] - g0[..., 1]
    gmdb = jnp.broadcast_to(gmd[:, :, None], (Bsz, N, N))
    u1 = jax.random.uniform(jax.random.fold_in(mkey, 1), (Bsz, N, N, 2),
                            minval=1e-10, maxval=1.0)
    g1 = -jnp.log(-jnp.log(u1) + 1e-10)
    gd = g1[..., 0] - g1[..., 1]

    ids = (jnp.argmax(action, axis=-1).astype(jnp.int32) + 1)

    pad = LP - L
    w1top = jnp.pad(W1[:, :D, :], ((0, 0), (0, 0), (0, pad)))
    w1bot = jnp.pad(W1[:, D:, :], ((0, 0), (0, 0), (0, pad)))
    w1cat = jnp.concatenate([w1top, w1bot], axis=-1)
    b1p = jnp.pad(b1, ((0, 0), (0, pad))).at[:, L].set(1.0)
    w2p = jnp.pad(W2[:, :, 0], ((0, 0), (0, pad))).at[:, L].set(b2[:, 0])

    w1c0 = w1cat[0]
    b10 = b1p[0:1]
    w20 = w2p[0:1]
    b1e = b1p[:, None, :]
    w2e = w2p[:, None, :]

    act3 = action[:, None, :].astype(f32)
    bm2 = bm[None, :]

    grid_spec = pltpu.PrefetchScalarGridSpec(
        num_scalar_prefetch=1,
        grid=(Bsz,),
        in_specs=[
            pl.BlockSpec((1, N, D), lambda i, ids: (i, 0, 0)),
            pl.BlockSpec((1, 1, A), lambda i, ids: (i, 0, 0)),
            pl.BlockSpec((A, D), lambda i, ids: (0, 0)),
            pl.BlockSpec((1, D), lambda i, ids: (0, 0)),
            pl.BlockSpec((1, N, N), lambda i, ids: (i, 0, 0)),
            pl.BlockSpec((1, N, N), lambda i, ids: (i, 0, 0)),
            pl.BlockSpec((D, 2 * LP), lambda i, ids: (0, 0)),
            pl.BlockSpec((1, LP), lambda i, ids: (0, 0)),
            pl.BlockSpec((1, LP), lambda i, ids: (0, 0)),
            pl.BlockSpec((1, D, 2 * LP), lambda i, ids: (ids[i], 0, 0)),
            pl.BlockSpec((1, 1, LP), lambda i, ids: (ids[i], 0, 0)),
            pl.BlockSpec((1, 1, LP), lambda i, ids: (ids[i], 0, 0)),
        ],
        out_specs=pl.BlockSpec((1, N, N), lambda i, ids: (i, 0, 0)),
    )

    import functools
    body = functools.partial(_ct_body, lp=LP)
    return pl.pallas_call(
        body,
        grid_spec=grid_spec,
        out_shape=jax.ShapeDtypeStruct((Bsz, N, N), f32),
    )(ids, x, act3, Wm, bm2, gmdb, gd, w1c0, b10, w20, w1cat, b1e, w2e)


# speedup vs baseline: 18.9549x; 1.5043x over previous
import functools

import jax
import jax.numpy as jnp
from jax.experimental import pallas as pl
from jax.experimental.pallas import tpu as pltpu

_LOGC = 9.210340371976182


def _ct_body(ids_ref, x_ref, act_ref, wm_ref, bm_ref, gmdb_ref, gd_ref,
             w10_ref, b10_ref, w20_ref, b20_ref,
             w1e_ref, b1e_ref, w2e_ref, b2e_ref,
             out_ref, s0_ref, *, d, n, l):
    xb = x_ref[0]

    im = jax.nn.sigmoid(
        jnp.dot(act_ref[0], wm_ref[...], preferred_element_type=jnp.float32)
        + bm_ref[...])
    v = jnp.sum(xb * im, axis=1, keepdims=True)
    ldm = (jnp.log(jnp.maximum(v, 1e-4))
           - jnp.log(jnp.maximum(1.0 - v, 1e-4)))
    mask2d = ldm > gmdb_ref[0]

    def escore(w1, b1r, w2b, b2r):
        a = jnp.dot(xb, w1[:d], preferred_element_type=jnp.float32) + b1r
        b = jnp.dot(xb, w1[d:], preferred_element_type=jnp.float32)
        h = a[:, None, :] + b[None, :, :]
        hb = jnp.maximum(h, 0.01 * h).astype(jnp.bfloat16)
        s = jax.lax.dot_general(hb.reshape(n * n, l), jnp.transpose(w2b),
                                (((1,), (0,)), ((), ())),
                                preferred_element_type=jnp.float32)
        return s.reshape(n, n) + b2r

    se = escore(w1e_ref[0], b1e_ref[0], w2e_ref[0], b2e_ref[0])

    @pl.when(jnp.logical_not(jnp.all(mask2d)))
    def _():
        s0_ref[...] = escore(w10_ref[...], b10_ref[...], w20_ref[...],
                             b20_ref[...])

    s = jnp.where(mask2d, se, s0_ref[...])
    ld = jnp.clip(s, -_LOGC, _LOGC)
    out_ref[0] = (ld > gd_ref[0]).astype(jnp.float32)


def kernel(x, action, Wm, bm, W1, b1, W2, b2):
    Bsz, N, D = x.shape
    A = action.shape[1]
    E, _, L = W1.shape
    f32 = jnp.float32

    mkey = jax.random.key(42)
    u0 = jax.random.uniform(jax.random.fold_in(mkey, 0), (Bsz, N, 2),
                            minval=1e-10, maxval=1.0)
    g0 = -jnp.log(-jnp.log(u0) + 1e-10)
    gmd = g0[..., 0] - g0[..., 1]
    gmdb = jnp.broadcast_to(gmd[:, :, None], (Bsz, N, N))
    u1 = jax.random.uniform(jax.random.fold_in(mkey, 1), (Bsz, N, N, 2),
                            minval=1e-10, maxval=1.0)
    g1 = -jnp.log(-jnp.log(u1) + 1e-10)
    gd = g1[..., 0] - g1[..., 1]

    ids = jnp.argmax(action, axis=-1).astype(jnp.int32) + 1

    b1a = b1[:, None, :]
    w2a = W2[:, :, 0][:, None, :].astype(jnp.bfloat16)
    b2a = jnp.broadcast_to(b2[:, :, None], (E, 1, N))

    w10 = W1[0]
    b10, w20, b20 = b1a[0], w2a[0], b2a[0]
    bm2 = bm[None, :]

    grid_spec = pltpu.PrefetchScalarGridSpec(
        num_scalar_prefetch=1,
        grid=(Bsz,),
        in_specs=[
            pl.BlockSpec((1, N, D), lambda i, ids: (i, 0, 0)),
            pl.BlockSpec((1, 1, A), lambda i, ids: (i, 0, 0)),
            pl.BlockSpec((A, D), lambda i, ids: (0, 0)),
            pl.BlockSpec((1, D), lambda i, ids: (0, 0)),
            pl.BlockSpec((1, N, N), lambda i, ids: (i, 0, 0)),
            pl.BlockSpec((1, N, N), lambda i, ids: (i, 0, 0)),
            pl.BlockSpec((2 * D, L), lambda i, ids: (0, 0)),
            pl.BlockSpec((1, L), lambda i, ids: (0, 0)),
            pl.BlockSpec((1, L), lambda i, ids: (0, 0)),
            pl.BlockSpec((1, N), lambda i, ids: (0, 0)),
            pl.BlockSpec((1, 2 * D, L), lambda i, ids: (ids[i], 0, 0)),
            pl.BlockSpec((1, 1, L), lambda i, ids: (ids[i], 0, 0)),
            pl.BlockSpec((1, 1, L), lambda i, ids: (ids[i], 0, 0)),
            pl.BlockSpec((1, 1, N), lambda i, ids: (ids[i], 0, 0)),
        ],
        out_specs=pl.BlockSpec((1, N, N), lambda i, ids: (i, 0, 0)),
        scratch_shapes=[pltpu.VMEM((N, N), f32)],
    )

    body = functools.partial(_ct_body, d=D, n=N, l=L)
    return pl.pallas_call(
        body,
        grid_spec=grid_spec,
        out_shape=jax.ShapeDtypeStruct((Bsz, N, N), f32),
    )(ids, x, action[:, None, :], Wm, bm2, gmdb, gd,
      w10, b10, w20, b20, W1, b1a, w2a, b2a)


# speedup vs baseline: 20.9601x; 1.1058x over previous
import functools

import jax
import jax.numpy as jnp
from jax.experimental import pallas as pl
from jax.experimental.pallas import tpu as pltpu

_LOGC = 9.210340371976182
_GRP = 4


def _ct_body(ids_ref, x_ref, act_ref, wm_ref, bm_ref, gmdb_ref, gd_ref,
             w10_ref, b10_ref, w20_ref, b20_ref, *rest, d, n, l):
    eref = rest[:4 * _GRP]
    out_ref, s0_ref = rest[4 * _GRP], rest[4 * _GRP + 1]

    def escore(xb, w1, b1r, w2b, b2r):
        a = jnp.dot(xb, w1[:d], preferred_element_type=jnp.float32) + b1r
        b = jnp.dot(xb, w1[d:], preferred_element_type=jnp.float32)
        h = a[:, None, :] + b[None, :, :]
        hb = jnp.maximum(h, 0.01 * h).astype(jnp.bfloat16)
        s = jax.lax.dot_general(hb.reshape(n * n, l), jnp.transpose(w2b),
                                (((1,), (0,)), ((), ())),
                                preferred_element_type=jnp.float32)
        return s.reshape(n, n) + b2r

    for g in range(_GRP):
        xb = x_ref[g]
        im = jax.nn.sigmoid(
            jnp.dot(act_ref[g], wm_ref[...],
                    preferred_element_type=jnp.float32)
            + bm_ref[...])
        v = jnp.sum(xb * im, axis=1, keepdims=True)
        ldm = (jnp.log(jnp.maximum(v, 1e-4))
               - jnp.log(jnp.maximum(1.0 - v, 1e-4)))
        mask2d = ldm > gmdb_ref[g]

        w1e, b1e, w2e, b2e = (eref[4 * g], eref[4 * g + 1],
                              eref[4 * g + 2], eref[4 * g + 3])
        se = escore(xb, w1e[0], b1e[0], w2e[0], b2e[0])

        @pl.when(jnp.logical_not(jnp.all(mask2d)))
        def _(xb=xb, g=g):
            s0_ref[g] = escore(xb, w10_ref[...], b10_ref[...],
                               w20_ref[...], b20_ref[...])

        s = jnp.where(mask2d, se, s0_ref[g])
        ld = jnp.clip(s, -_LOGC, _LOGC)
        out_ref[g] = (ld > gd_ref[g]).astype(jnp.float32)


def kernel(x, action, Wm, bm, W1, b1, W2, b2):
    Bsz, N, D = x.shape
    A = action.shape[1]
    E, _, L = W1.shape
    G = _GRP
    f32 = jnp.float32

    mkey = jax.random.key(42)
    u0 = jax.random.uniform(jax.random.fold_in(mkey, 0), (Bsz, N, 2),
                            minval=1e-10, maxval=1.0)
    g0 = -jnp.log(-jnp.log(u0) + 1e-10)
    gmd = g0[..., 0] - g0[..., 1]
    gmdb = jnp.broadcast_to(gmd[:, :, None], (Bsz, N, N))
    u1 = jax.random.uniform(jax.random.fold_in(mkey, 1), (Bsz, N, N, 2),
                            minval=1e-10, maxval=1.0)
    g1 = -jnp.log(-jnp.log(u1) + 1e-10)
    gd = g1[..., 0] - g1[..., 1]

    ids = jnp.argmax(action, axis=-1).astype(jnp.int32) + 1

    b1a = b1[:, None, :]
    w2a = W2[:, :, 0][:, None, :].astype(jnp.bfloat16)
    b2a = jnp.broadcast_to(b2[:, :, None], (E, 1, N))

    w10 = W1[0]
    b10, w20, b20 = b1a[0], w2a[0], b2a[0]
    bm2 = bm[None, :]

    base_specs = [
        pl.BlockSpec((G, N, D), lambda i, ids: (i, 0, 0)),
        pl.BlockSpec((G, 1, A), lambda i, ids: (i, 0, 0)),
        pl.BlockSpec((A, D), lambda i, ids: (0, 0)),
        pl.BlockSpec((1, D), lambda i, ids: (0, 0)),
        pl.BlockSpec((G, N, N), lambda i, ids: (i, 0, 0)),
        pl.BlockSpec((G, N, N), lambda i, ids: (i, 0, 0)),
        pl.BlockSpec((2 * D, L), lambda i, ids: (0, 0)),
        pl.BlockSpec((1, L), lambda i, ids: (0, 0)),
        pl.BlockSpec((1, L), lambda i, ids: (0, 0)),
        pl.BlockSpec((1, N), lambda i, ids: (0, 0)),
    ]
    esp = []
    for g in range(G):
        esp += [
            pl.BlockSpec((1, 2 * D, L),
                         lambda i, ids, g=g: (ids[G * i + g], 0, 0)),
            pl.BlockSpec((1, 1, L),
                         lambda i, ids, g=g: (ids[G * i + g], 0, 0)),
            pl.BlockSpec((1, 1, L),
                         lambda i, ids, g=g: (ids[G * i + g], 0, 0)),
            pl.BlockSpec((1, 1, N),
                         lambda i, ids, g=g: (ids[G * i + g], 0, 0)),
        ]

    grid_spec = pltpu.PrefetchScalarGridSpec(
        num_scalar_prefetch=1,
        grid=(Bsz // G,),
        in_specs=base_specs + esp,
        out_specs=pl.BlockSpec((G, N, N), lambda i, ids: (i, 0, 0)),
        scratch_shapes=[pltpu.VMEM((G, N, N), f32)],
    )

    eargs = []
    for g in range(G):
        eargs += [W1, b1a, w2a, b2a]

    body = functools.partial(_ct_body, d=D, n=N, l=L)
    return pl.pallas_call(
        body,
        grid_spec=grid_spec,
        out_shape=jax.ShapeDtypeStruct((Bsz, N, N), f32),
    )(ids, x, action[:, None, :], Wm, bm2, gmdb, gd,
      w10, b10, w20, b20, *eargs)
